# baseline (speedup 1.0000x reference)
import functools

import jax
import jax.numpy as jnp
from jax import lax
from jax.experimental import pallas as pl
from jax.experimental.pallas import tpu as pltpu
from jax.experimental.pallas import tpu_sc as plsc

N = 10000
NT = 10240
E = 320000
DH = 20
DP = 24
NCLS = 10
BN_EPS = 1e-5

NC = 2
NS = 16
EPT = 20480
EPADC = NS * EPT
CHUNK = 256
NSUB = CHUNK // 128
NCHUNK = EPT // CHUNK
G = CHUNK // 16
RPT = NT // NS
INV_SQRT_D = 1.0 / (DH ** 0.5)


def _write_tables(y, qt_ref, kt_ref, vt_ref, skip_ref):
    rows = y.shape[0]
    z = jnp.zeros((rows, DP - DH), jnp.float32)
    qt_ref[0] = jnp.concatenate([y[:, 0:20], z], axis=1)
    qt_ref[1] = jnp.concatenate([y[:, 80:100], z], axis=1)
    kt_ref[0] = jnp.concatenate([y[:, 20:40], z], axis=1)
    kt_ref[1] = jnp.concatenate([y[:, 100:120], z], axis=1)
    vt_ref[0] = jnp.concatenate([y[:, 40:60], z], axis=1)
    vt_ref[1] = jnp.concatenate([y[:, 120:140], z], axis=1)
    skip_ref[...] = jnp.concatenate([y[:, 60:80], y[:, 140:160]], axis=1)


def _proj0_body(x_ref, w_ref, b_ref, qt_ref, kt_ref, vt_ref, skip_ref):
    y = jnp.dot(x_ref[...], w_ref[...], preferred_element_type=jnp.float32)
    y = y + b_ref[...]
    _write_tables(y, qt_ref, kt_ref, vt_ref, skip_ref)


def _combine(agg_ref, s_ref, skip_ref, scl_ref, ab_ref):
    a1 = ab_ref[0, 0]
    a2 = ab_ref[0, 1]
    s1 = s_ref[0, :][:, None] + 1e-16
    s2 = s_ref[1, :][:, None] + 1e-16
    x = (a1 * (agg_ref[0, :, 0:20] / s1 + skip_ref[:, 0:20])
         + a2 * (agg_ref[1, :, 0:20] / s2 + skip_ref[:, 20:40]))
    x = x * scl_ref[0:1, :] + scl_ref[1:2, :]
    return jnp.where(x >= 0, x, 0.01 * x)


def _mid_body(agg_ref, s_ref, skip_ref, scl_ref, ab_ref, w_ref, b_ref,
              x_ref, qt_ref, kt_ref, vt_ref, skip_o_ref):
    x = _combine(agg_ref, s_ref, skip_ref, scl_ref, ab_ref)
    x_ref[...] = x
    y = jnp.dot(x, w_ref[...], preferred_element_type=jnp.float32) + b_ref[...]
    _write_tables(y, qt_ref, kt_ref, vt_ref, skip_o_ref)


def _final_body(agg_ref, s_ref, skip_ref, scl_ref, ab_ref,
                x0_ref, x1_ref, x2_ref, wout_ref, bout_ref, out_ref):
    x3 = _combine(agg_ref, s_ref, skip_ref, scl_ref, ab_ref)
    fc = jnp.concatenate([x0_ref[...], x1_ref[...], x2_ref[...], x3], axis=1)
    out_ref[...] = (jnp.dot(fc, wout_ref[...],
                            preferred_element_type=jnp.float32)
                    + bout_ref[...])


_RB = 1024
_NRB = NT // _RB

_full = lambda shape: pl.BlockSpec(shape, lambda i: tuple(0 for _ in shape))
_rows = lambda width: pl.BlockSpec((_RB, width), lambda i: (i, 0))
_tab = pl.BlockSpec((2, _RB, DP), lambda i: (0, i, 0))
_svec = pl.BlockSpec((2, _RB), lambda i: (0, i))

_TAB_OUT = [jax.ShapeDtypeStruct((2, NT, DP), jnp.float32)] * 3

_proj0 = pl.pallas_call(
    _proj0_body,
    grid=(_NRB,),
    in_specs=[_rows(128), _full((128, 160)), _full((1, 160))],
    out_specs=[_tab, _tab, _tab, _rows(40)],
    out_shape=_TAB_OUT + [jax.ShapeDtypeStruct((NT, 40), jnp.float32)],
)

_mid = pl.pallas_call(
    _mid_body,
    grid=(_NRB,),
    in_specs=[_tab, _svec, _rows(40), _full((2, 20)), _full((1, 2)),
              _full((20, 160)), _full((1, 160))],
    out_specs=[_rows(20), _tab, _tab, _tab, _rows(40)],
    out_shape=[jax.ShapeDtypeStruct((NT, 20), jnp.float32)] + _TAB_OUT
    + [jax.ShapeDtypeStruct((NT, 40), jnp.float32)],
)

_final = pl.pallas_call(
    _final_body,
    grid=(_NRB,),
    in_specs=[_tab, _svec, _rows(40), _full((2, 20)), _full((1, 2)),
              _rows(20), _rows(20), _rows(20),
              _full((4 * DH, NCLS)), _full((1, NCLS))],
    out_specs=_rows(NCLS),
    out_shape=jax.ShapeDtypeStruct((NT, NCLS), jnp.float32),
)


def _sc_conv_body(qt, kt, vt, eidx, out, out_s,
                  qrows, krows, vrows, srows, idxb, idq, idk, dloc, dsct,
                  s_loc, tmp_s, acc_s,
                  agg_sh, s_stage,
                  gq, gk, gv, ix, sct):
    cid = lax.axis_index("c")
    sid = lax.axis_index("s")
    iota = lax.iota(jnp.int32, 16)
    zero16 = jnp.zeros((16,), jnp.float32)
    tbase = jnp.full((16,), 0, jnp.int32) + cid * NT
    perms = [(iota + sh) % 16 for sh in (8, 4, 2, 1)]
    m8 = iota >= 8

    def fire_idx(ci, par):
        pltpu.async_copy(eidx.at[cid, sid, ci], idxb[par], ix[par])

    def wait_idx(par):
        pltpu.make_async_copy(eidx.at[cid, sid, 0], idxb[par],
                              ix[par]).wait()

    def extract(par):
        for r in range(NSUB):
            for t in range(8):
                s = pl.ds(t * 16, 16)
                idk[par][r, s] = idxb[par][r, s] + tbase
                d = idxb[par][NSUB + r, s]
                idq[par][r, s] = d + tbase
                dloc[par][r, s] = d

    def fire_gathers(par):
        for j in range(NSUB):
            sl = pl.ds(j * 128, 128)
            pltpu.async_copy(qt.at[idq[par].at[j]], qrows[par].at[sl], gq[par])
            pltpu.async_copy(kt.at[idk[par].at[j]], krows[par].at[sl], gk[par])
            pltpu.async_copy(vt.at[idk[par].at[j]], vrows[par].at[sl], gv[par])

    def wait_gathers(par):
        for j in range(NSUB):
            sl = pl.ds(j * 128, 128)
            pltpu.make_async_copy(qt.at[idq[par].at[j]], qrows[par].at[sl],
                                  gq[par]).wait()
            pltpu.make_async_copy(kt.at[idk[par].at[j]], krows[par].at[sl],
                                  gk[par]).wait()
            pltpu.make_async_copy(vt.at[idk[par].at[j]], vrows[par].at[sl],
                                  gv[par]).wait()

    def fire_scat(par):
        for j in range(NSUB):
            pltpu.async_copy(srows[par].at[pl.ds(j * 128, 128)],
                             agg_sh.at[dsct[par].at[j]], sct[par], add=True)

    def wait_scat(par):
        for j in range(NSUB):
            pltpu.make_async_copy(srows[par].at[pl.ds(j * 128, 128)],
                                  agg_sh.at[dsct[par].at[j]],
                                  sct[par]).wait()

    def _zs(i, c):
        s_loc[pl.ds(i * 16, 16)] = zero16
        return c
    lax.fori_loop(0, NT // 16, _zs, 0)

    def _zq(i, c):
        qrows[0][i, pl.ds(0, 16)] = zero16
        qrows[0][i, pl.ds(8, 16)] = zero16
        return c
    lax.fori_loop(0, CHUNK, _zq, 0)
    for o in range(0, RPT, CHUNK):
        n = min(CHUNK, RPT - o)
        pltpu.sync_copy(qrows[0].at[pl.ds(0, n)],
                        agg_sh.at[pl.ds(sid * RPT + o, n)])

    pltpu.sync_copy(eidx.at[cid, sid, 0], idxb[0])
    extract(0)
    fire_gathers(0)
    fire_idx(1, 1)

    def _chunk(c, par):
        nxt = c + 1

        @pl.when(nxt < NCHUNK)
        def _():
            wait_idx(1 - par)
            extract(1 - par)
            fire_gathers(1 - par)

        wait_gathers(par)

        @pl.when(c + 2 < NCHUNK)
        def _():
            fire_idx(c + 2, par)

        @pl.when(c >= 2)
        def _():
            wait_scat(par)

        def _grp(g, cc):
            base_r = g * 16
            e_vec = jnp.zeros((16,), jnp.float32)
            for j in range(16):
                r = base_r + j
                vhiprod = (qrows[par][r, pl.ds(8, 16)]
                           * krows[par][r, pl.ds(8, 16)])
                v = (qrows[par][r, pl.ds(0, 16)] * krows[par][r, pl.ds(0, 16)]
                     + jnp.where(m8, vhiprod, 0.0))
                for p in perms:
                    v = v + jnp.take(v, p)
                e_vec = jnp.where(iota == j, v, e_vec)
            e = jnp.exp(e_vec * INV_SQRT_D)
            d16 = dloc[par][g // 8, pl.ds((g % 8) * 16, 16)]
            plsc.addupdate_scatter(s_loc, [d16], e)
            for j in range(16):
                r = base_r + j
                e_b = jnp.take(e, jnp.full((16,), j, jnp.int32))
                srows[par][r, pl.ds(0, 16)] = (
                    vrows[par][r, pl.ds(0, 16)] * e_b)
                srows[par][r, pl.ds(8, 16)] = (
                    vrows[par][r, pl.ds(8, 16)] * e_b)
            return cc
        lax.fori_loop(0, G, _grp, 0)
        for r in range(NSUB):
            for t in range(8):
                s = pl.ds(t * 16, 16)
                dsct[par][r, s] = dloc[par][r, s]
        fire_scat(par)
        return c + 1

    def _loop(i, c):
        c = _chunk(c, 0)
        c = _chunk(c, 1)
        return c
    lax.fori_loop(0, NCHUNK // 2, _loop, 0)
    wait_scat(0)
    wait_scat(1)

    pltpu.sync_copy(s_loc, s_stage.at[sid])
    plsc.subcore_barrier()

    SSL = NT // NS

    def _za(i, c):
        acc_s[pl.ds(i * 16, 16)] = zero16
        return c
    lax.fori_loop(0, SSL // 16, _za, 0)

    def _red(j, c):
        pltpu.sync_copy(s_stage.at[j, pl.ds(sid * SSL, SSL)], tmp_s)

        def _add(g, cc):
            acc_s[pl.ds(g * 16, 16)] = (acc_s[pl.ds(g * 16, 16)]
                                        + tmp_s[pl.ds(g * 16, 16)])
            return cc
        lax.fori_loop(0, SSL // 16, _add, 0)
        return c
    lax.fori_loop(0, NS, _red, 0)
    pltpu.sync_copy(acc_s, out_s.at[cid, pl.ds(sid * SSL, SSL)])
    pltpu.sync_copy(agg_sh.at[pl.ds(sid * RPT, RPT)],
                    out.at[cid, pl.ds(sid * RPT, RPT)])


def _build_sc_conv(interpret=False):
    return functools.partial(
        pl.kernel,
        out_type=[jax.ShapeDtypeStruct((2, NT, DP), jnp.float32),
                  jax.ShapeDtypeStruct((2, NT), jnp.float32)],
        mesh=plsc.VectorSubcoreMesh(core_axis_name="c", subcore_axis_name="s"),
        interpret=interpret,
        compiler_params=pltpu.CompilerParams(
            use_tc_tiling_on_sc=False, needs_layout_passes=False),
        scratch_types=[
            [pltpu.VMEM((CHUNK, DP), jnp.float32)] * 2,
            [pltpu.VMEM((CHUNK, DP), jnp.float32)] * 2,
            [pltpu.VMEM((CHUNK, DP), jnp.float32)] * 2,
            [pltpu.VMEM((CHUNK, DP), jnp.float32)] * 2,
            [pltpu.VMEM((2 * NSUB, 128), jnp.int32)] * 2,
            [pltpu.VMEM((NSUB, 128), jnp.int32)] * 2,
            [pltpu.VMEM((NSUB, 128), jnp.int32)] * 2,
            [pltpu.VMEM((NSUB, 128), jnp.int32)] * 2,
            [pltpu.VMEM((NSUB, 128), jnp.int32)] * 2,
            pltpu.VMEM((NT,), jnp.float32),
            pltpu.VMEM((NT // NS,), jnp.float32),
            pltpu.VMEM((NT // NS,), jnp.float32),
            pltpu.VMEM_SHARED((NT, DP), jnp.float32),
            pltpu.VMEM_SHARED((NS, NT), jnp.float32),
            [pltpu.SemaphoreType.DMA] * 2,
            [pltpu.SemaphoreType.DMA] * 2,
            [pltpu.SemaphoreType.DMA] * 2,
            [pltpu.SemaphoreType.DMA] * 2,
            [pltpu.SemaphoreType.DMA] * 2,
        ],
    )(_sc_conv_body)


_sc_conv = _build_sc_conv()


def _edge_arrays(same_index, diff_index):
    pad = EPADC - E
    pad_node = (jnp.arange(pad, dtype=jnp.int32) * 37) % N
    pad_trash = N + (jnp.arange(pad, dtype=jnp.int32) % (NT - N))

    def build(idx):
        src = jnp.concatenate([idx[0].astype(jnp.int32), pad_node])
        dst = jnp.concatenate([idx[1].astype(jnp.int32), pad_trash])
        src = src.reshape(NS, NCHUNK, NSUB, 128)
        dst = dst.reshape(NS, NCHUNK, NSUB, 128)
        return jnp.concatenate([src, dst], axis=2)

    return jnp.stack([build(same_index), build(diff_index)])


def _layer_params(params, l):
    c1 = params['convs1'][l]
    c2 = params['convs2'][l]
    wcat = jnp.concatenate(
        [c1['Wq'], c1['Wk'], c1['Wv'], c1['Ws'],
         c2['Wq'], c2['Wk'], c2['Wv'], c2['Ws']], axis=1)
    bcat = jnp.concatenate(
        [c1['bq'], c1['bk'], c1['bv'], c1['bs'],
         c2['bq'], c2['bk'], c2['bv'], c2['bs']])[None, :]
    return wcat, bcat


def _affine_params(params, l):
    scale = params['bn_gamma'][l] / jnp.sqrt(1.0 + BN_EPS)
    scl = jnp.stack([scale, params['bn_beta'][l]])
    tot = params['weights1'][l] + params['weights2'][l]
    ab = jnp.stack([params['weights1'][l] / tot,
                    params['weights2'][l] / tot])[None, :]
    return scl, ab


def kernel(features, same_index, diff_index, params):
    eidx = _edge_arrays(same_index, diff_index)
    xp = jnp.pad(features, ((0, NT - N), (0, 0)))

    w0, b0 = _layer_params(params, 0)
    qt, kt, vt, skip = _proj0(xp, w0, b0)
    flat = lambda t: t.reshape(2 * NT, DP)
    agg, sden = _sc_conv(flat(qt), flat(kt), flat(vt), eidx)

    xs = []
    for l in range(1, 4):
        wl, bl = _layer_params(params, l)
        scl, ab = _affine_params(params, l - 1)
        x_prev, qt, kt, vt, skip = _mid(agg, sden, skip, scl, ab, wl, bl)
        xs.append(x_prev)
        agg, sden = _sc_conv(flat(qt), flat(kt), flat(vt), eidx)

    scl, ab = _affine_params(params, 3)
    out = _final(agg, sden, skip, scl, ab, xs[0], xs[1], xs[2],
                 params['Wout'], params['bout'][None, :])
    return out[:N]


# speedup vs baseline: 1.0242x; 1.0242x over previous
import functools

import jax
import jax.numpy as jnp
from jax import lax
from jax.experimental import pallas as pl
from jax.experimental.pallas import tpu as pltpu
from jax.experimental.pallas import tpu_sc as plsc

N = 10000
NT = 10240
E = 320000
DH = 20
DP = 32
NCLS = 10
BN_EPS = 1e-5

NC = 2
NS = 16
EPT = 20480
EPADC = NS * EPT
CHUNK = 256
NSUB = CHUNK // 128
NCHUNK = EPT // CHUNK
G = CHUNK // 16
RPT = NT // NS
INV_SQRT_D = 1.0 / (DH ** 0.5)


def _write_tables(y, qt_ref, kt_ref, vt_ref, skip_ref):
    rows = y.shape[0]
    z = jnp.zeros((rows, DP - DH), jnp.float32)
    qt_ref[0] = jnp.concatenate([y[:, 0:20], z], axis=1)
    qt_ref[1] = jnp.concatenate([y[:, 80:100], z], axis=1)
    kt_ref[0] = jnp.concatenate([y[:, 20:40], z], axis=1)
    kt_ref[1] = jnp.concatenate([y[:, 100:120], z], axis=1)
    vt_ref[0] = jnp.concatenate([y[:, 40:60], z], axis=1)
    vt_ref[1] = jnp.concatenate([y[:, 120:140], z], axis=1)
    skip_ref[...] = jnp.concatenate([y[:, 60:80], y[:, 140:160]], axis=1)


def _proj0_body(x_ref, w_ref, b_ref, qt_ref, kt_ref, vt_ref, skip_ref):
    y = jnp.dot(x_ref[...], w_ref[...], preferred_element_type=jnp.float32)
    y = y + b_ref[...]
    _write_tables(y, qt_ref, kt_ref, vt_ref, skip_ref)


def _combine(agg_ref, s_ref, skip_ref, scl_ref, ab_ref):
    a1 = ab_ref[0, 0]
    a2 = ab_ref[0, 1]
    s1 = s_ref[0, :][:, None] + 1e-16
    s2 = s_ref[1, :][:, None] + 1e-16
    x = (a1 * (agg_ref[0, :, 0:20] / s1 + skip_ref[:, 0:20])
         + a2 * (agg_ref[1, :, 0:20] / s2 + skip_ref[:, 20:40]))
    x = x * scl_ref[0:1, :] + scl_ref[1:2, :]
    return jnp.where(x >= 0, x, 0.01 * x)


def _mid_body(agg_ref, s_ref, skip_ref, scl_ref, ab_ref, w_ref, b_ref,
              x_ref, qt_ref, kt_ref, vt_ref, skip_o_ref):
    x = _combine(agg_ref, s_ref, skip_ref, scl_ref, ab_ref)
    x_ref[...] = x
    y = jnp.dot(x, w_ref[...], preferred_element_type=jnp.float32) + b_ref[...]
    _write_tables(y, qt_ref, kt_ref, vt_ref, skip_o_ref)


def _final_body(agg_ref, s_ref, skip_ref, scl_ref, ab_ref,
                x0_ref, x1_ref, x2_ref, wout_ref, bout_ref, out_ref):
    x3 = _combine(agg_ref, s_ref, skip_ref, scl_ref, ab_ref)
    fc = jnp.concatenate([x0_ref[...], x1_ref[...], x2_ref[...], x3], axis=1)
    out_ref[...] = (jnp.dot(fc, wout_ref[...],
                            preferred_element_type=jnp.float32)
                    + bout_ref[...])


_RB = 1024
_NRB = NT // _RB

_full = lambda shape: pl.BlockSpec(shape, lambda i: tuple(0 for _ in shape))
_rows = lambda width: pl.BlockSpec((_RB, width), lambda i: (i, 0))
_tab = pl.BlockSpec((2, _RB, DP), lambda i: (0, i, 0))
_svec = pl.BlockSpec((2, _RB), lambda i: (0, i))

_TAB_OUT = [jax.ShapeDtypeStruct((2, NT, DP), jnp.float32)] * 3

_proj0 = pl.pallas_call(
    _proj0_body,
    grid=(_NRB,),
    in_specs=[_rows(128), _full((128, 160)), _full((1, 160))],
    out_specs=[_tab, _tab, _tab, _rows(40)],
    out_shape=_TAB_OUT + [jax.ShapeDtypeStruct((NT, 40), jnp.float32)],
)

_mid = pl.pallas_call(
    _mid_body,
    grid=(_NRB,),
    in_specs=[_tab, _svec, _rows(40), _full((2, 20)), _full((1, 2)),
              _full((20, 160)), _full((1, 160))],
    out_specs=[_rows(20), _tab, _tab, _tab, _rows(40)],
    out_shape=[jax.ShapeDtypeStruct((NT, 20), jnp.float32)] + _TAB_OUT
    + [jax.ShapeDtypeStruct((NT, 40), jnp.float32)],
)

_final = pl.pallas_call(
    _final_body,
    grid=(_NRB,),
    in_specs=[_tab, _svec, _rows(40), _full((2, 20)), _full((1, 2)),
              _rows(20), _rows(20), _rows(20),
              _full((4 * DH, NCLS)), _full((1, NCLS))],
    out_specs=_rows(NCLS),
    out_shape=jax.ShapeDtypeStruct((NT, NCLS), jnp.float32),
)


def _sc_conv_body(qt, kt, vt, eidx, out, out_s,
                  qrows, krows, vrows, srows, idxb, idq, idk, dloc, dsct,
                  s_loc, tmp_s, acc_s,
                  agg_sh, s_stage,
                  gq, gk, gv, ix, sct):
    cid = lax.axis_index("c")
    sid = lax.axis_index("s")
    iota = lax.iota(jnp.int32, 16)
    zero16 = jnp.zeros((16,), jnp.float32)
    tbase = jnp.full((16,), 0, jnp.int32) + cid * NT
    perms = [(iota + sh) % 16 for sh in (8, 4, 2, 1)]

    def fire_idx(ci, par):
        pltpu.async_copy(eidx.at[cid, sid, ci], idxb[par], ix[par])

    def wait_idx(par):
        pltpu.make_async_copy(eidx.at[cid, sid, 0], idxb[par],
                              ix[par]).wait()

    def extract(par):
        for r in range(NSUB):
            for t in range(8):
                s = pl.ds(t * 16, 16)
                idk[par][r, s] = idxb[par][r, s] + tbase
                d = idxb[par][NSUB + r, s]
                idq[par][r, s] = d + tbase
                dloc[par][r, s] = d

    def fire_gathers(par):
        for j in range(NSUB):
            sl = pl.ds(j * 128, 128)
            pltpu.async_copy(qt.at[idq[par].at[j]], qrows[par].at[sl], gq[par])
            pltpu.async_copy(kt.at[idk[par].at[j]], krows[par].at[sl], gk[par])
            pltpu.async_copy(vt.at[idk[par].at[j]], vrows[par].at[sl], gv[par])

    def wait_gathers(par):
        for j in range(NSUB):
            sl = pl.ds(j * 128, 128)
            pltpu.make_async_copy(qt.at[idq[par].at[j]], qrows[par].at[sl],
                                  gq[par]).wait()
            pltpu.make_async_copy(kt.at[idk[par].at[j]], krows[par].at[sl],
                                  gk[par]).wait()
            pltpu.make_async_copy(vt.at[idk[par].at[j]], vrows[par].at[sl],
                                  gv[par]).wait()

    def fire_scat(par):
        for j in range(NSUB):
            pltpu.async_copy(srows[par].at[pl.ds(j * 128, 128)],
                             agg_sh.at[dsct[par].at[j]], sct[par], add=True)

    def wait_scat(par):
        for j in range(NSUB):
            pltpu.make_async_copy(srows[par].at[pl.ds(j * 128, 128)],
                                  agg_sh.at[dsct[par].at[j]],
                                  sct[par]).wait()

    def _zs(i, c):
        s_loc[pl.ds(i * 16, 16)] = zero16
        return c
    lax.fori_loop(0, NT // 16, _zs, 0)

    def _zq(i, c):
        qrows[0][i, pl.ds(0, 16)] = zero16
        qrows[0][i, pl.ds(16, 16)] = zero16
        return c
    lax.fori_loop(0, CHUNK, _zq, 0)
    for o in range(0, RPT, CHUNK):
        n = min(CHUNK, RPT - o)
        pltpu.sync_copy(qrows[0].at[pl.ds(0, n)],
                        agg_sh.at[pl.ds(sid * RPT + o, n)])

    pltpu.sync_copy(eidx.at[cid, sid, 0], idxb[0])
    extract(0)
    fire_gathers(0)
    fire_idx(1, 1)

    def _chunk(c, par):
        nxt = c + 1

        @pl.when(nxt < NCHUNK)
        def _():
            wait_idx(1 - par)
            extract(1 - par)
            fire_gathers(1 - par)

        wait_gathers(par)

        @pl.when(c + 2 < NCHUNK)
        def _():
            fire_idx(c + 2, par)

        @pl.when(c >= 2)
        def _():
            wait_scat(par)

        def _grp(g, cc):
            base_r = g * 16
            e_vec = jnp.zeros((16,), jnp.float32)
            for j in range(16):
                r = base_r + j
                v = (qrows[par][r, pl.ds(0, 16)] * krows[par][r, pl.ds(0, 16)]
                     + qrows[par][r, pl.ds(16, 16)]
                     * krows[par][r, pl.ds(16, 16)])
                for p in perms:
                    v = v + jnp.take(v, p)
                e_vec = jnp.where(iota == j, v, e_vec)
            e = jnp.exp(e_vec * INV_SQRT_D)
            d16 = dloc[par][g // 8, pl.ds((g % 8) * 16, 16)]
            plsc.addupdate_scatter(s_loc, [d16], e)
            for j in range(16):
                r = base_r + j
                e_b = jnp.take(e, jnp.full((16,), j, jnp.int32))
                srows[par][r, pl.ds(0, 16)] = (
                    vrows[par][r, pl.ds(0, 16)] * e_b)
                srows[par][r, pl.ds(16, 16)] = (
                    vrows[par][r, pl.ds(16, 16)] * e_b)
            return cc
        lax.fori_loop(0, G, _grp, 0)
        for r in range(NSUB):
            for t in range(8):
                s = pl.ds(t * 16, 16)
                dsct[par][r, s] = dloc[par][r, s]
        fire_scat(par)
        return c + 1

    def _loop(i, c):
        c = _chunk(c, 0)
        c = _chunk(c, 1)
        return c
    lax.fori_loop(0, NCHUNK // 2, _loop, 0)
    wait_scat(0)
    wait_scat(1)

    pltpu.sync_copy(s_loc, s_stage.at[sid])
    plsc.subcore_barrier()

    SSL = NT // NS

    def _za(i, c):
        acc_s[pl.ds(i * 16, 16)] = zero16
        return c
    lax.fori_loop(0, SSL // 16, _za, 0)

    def _red(j, c):
        pltpu.sync_copy(s_stage.at[j, pl.ds(sid * SSL, SSL)], tmp_s)

        def _add(g, cc):
            acc_s[pl.ds(g * 16, 16)] = (acc_s[pl.ds(g * 16, 16)]
                                        + tmp_s[pl.ds(g * 16, 16)])
            return cc
        lax.fori_loop(0, SSL // 16, _add, 0)
        return c
    lax.fori_loop(0, NS, _red, 0)
    pltpu.sync_copy(acc_s, out_s.at[cid, pl.ds(sid * SSL, SSL)])
    pltpu.sync_copy(agg_sh.at[pl.ds(sid * RPT, RPT)],
                    out.at[cid, pl.ds(sid * RPT, RPT)])


def _build_sc_conv(interpret=False):
    return functools.partial(
        pl.kernel,
        out_type=[jax.ShapeDtypeStruct((2, NT, DP), jnp.float32),
                  jax.ShapeDtypeStruct((2, NT), jnp.float32)],
        mesh=plsc.VectorSubcoreMesh(core_axis_name="c", subcore_axis_name="s"),
        interpret=interpret,
        compiler_params=pltpu.CompilerParams(
            use_tc_tiling_on_sc=False, needs_layout_passes=False),
        scratch_types=[
            [pltpu.VMEM((CHUNK, DP), jnp.float32)] * 2,
            [pltpu.VMEM((CHUNK, DP), jnp.float32)] * 2,
            [pltpu.VMEM((CHUNK, DP), jnp.float32)] * 2,
            [pltpu.VMEM((CHUNK, DP), jnp.float32)] * 2,
            [pltpu.VMEM((2 * NSUB, 128), jnp.int32)] * 2,
            [pltpu.VMEM((NSUB, 128), jnp.int32)] * 2,
            [pltpu.VMEM((NSUB, 128), jnp.int32)] * 2,
            [pltpu.VMEM((NSUB, 128), jnp.int32)] * 2,
            [pltpu.VMEM((NSUB, 128), jnp.int32)] * 2,
            pltpu.VMEM((NT,), jnp.float32),
            pltpu.VMEM((NT // NS,), jnp.float32),
            pltpu.VMEM((NT // NS,), jnp.float32),
            pltpu.VMEM_SHARED((NT, DP), jnp.float32),
            pltpu.VMEM_SHARED((NS, NT), jnp.float32),
            [pltpu.SemaphoreType.DMA] * 2,
            [pltpu.SemaphoreType.DMA] * 2,
            [pltpu.SemaphoreType.DMA] * 2,
            [pltpu.SemaphoreType.DMA] * 2,
            [pltpu.SemaphoreType.DMA] * 2,
        ],
    )(_sc_conv_body)


_sc_conv = _build_sc_conv()


def _edge_arrays(same_index, diff_index):
    pad = EPADC - E
    pad_node = (jnp.arange(pad, dtype=jnp.int32) * 37) % N
    pad_trash = N + (jnp.arange(pad, dtype=jnp.int32) % (NT - N))

    def build(idx):
        src = jnp.concatenate([idx[0].astype(jnp.int32), pad_node])
        dst = jnp.concatenate([idx[1].astype(jnp.int32), pad_trash])
        src = src.reshape(NS, NCHUNK, NSUB, 128)
        dst = dst.reshape(NS, NCHUNK, NSUB, 128)
        return jnp.concatenate([src, dst], axis=2)

    return jnp.stack([build(same_index), build(diff_index)])


def _layer_params(params, l):
    c1 = params['convs1'][l]
    c2 = params['convs2'][l]
    wcat = jnp.concatenate(
        [c1['Wq'], c1['Wk'], c1['Wv'], c1['Ws'],
         c2['Wq'], c2['Wk'], c2['Wv'], c2['Ws']], axis=1)
    bcat = jnp.concatenate(
        [c1['bq'], c1['bk'], c1['bv'], c1['bs'],
         c2['bq'], c2['bk'], c2['bv'], c2['bs']])[None, :]
    return wcat, bcat


def _affine_params(params, l):
    scale = params['bn_gamma'][l] / jnp.sqrt(1.0 + BN_EPS)
    scl = jnp.stack([scale, params['bn_beta'][l]])
    tot = params['weights1'][l] + params['weights2'][l]
    ab = jnp.stack([params['weights1'][l] / tot,
                    params['weights2'][l] / tot])[None, :]
    return scl, ab


def kernel(features, same_index, diff_index, params):
    eidx = _edge_arrays(same_index, diff_index)
    xp = jnp.pad(features, ((0, NT - N), (0, 0)))

    w0, b0 = _layer_params(params, 0)
    qt, kt, vt, skip = _proj0(xp, w0, b0)
    flat = lambda t: t.reshape(2 * NT, DP)
    agg, sden = _sc_conv(flat(qt), flat(kt), flat(vt), eidx)

    xs = []
    for l in range(1, 4):
        wl, bl = _layer_params(params, l)
        scl, ab = _affine_params(params, l - 1)
        x_prev, qt, kt, vt, skip = _mid(agg, sden, skip, scl, ab, wl, bl)
        xs.append(x_prev)
        agg, sden = _sc_conv(flat(qt), flat(kt), flat(vt), eidx)

    scl, ab = _affine_params(params, 3)
    out = _final(agg, sden, skip, scl, ab, xs[0], xs[1], xs[2],
                 params['Wout'], params['bout'][None, :])
    return out[:N]
